# baseline (speedup 1.0000x reference)
import jax
import jax.numpy as jnp
from jax.experimental import pallas as pl
from jax.experimental.pallas import tpu as pltpu

_B, _C, _N, _CL, _K = 4, 128, 10000, 256, 6
_NEG = float("-inf")
_BN_S = 0.9999950000374997


def _affine(x, s0, s1, g_ref, be_ref):
    mean = s0 * (1.0 / _N)
    var = s1 * (1.0 / _N) - mean * mean
    a = g_ref[...] * _BN_S * jax.lax.rsqrt(var + 1e-3)
    d = be_ref[...] - a * mean
    return jnp.maximum(a * x + d, 0.0)


_TC = 2048
_CHUNKS = [(o, min(_TC, _N - o)) for o in range(0, _N, _TC)]


def _body(x_ref, gd_ref, bd_ref, gu_ref, beu_ref, wd_ref, wu_ref, bu_ref,
          m1_ref, a2_ref, wg2t_ref, b1_ref, b2_ref, ws1_ref, ws2_ref, bs_ref,
          out_ref):
    s0 = jnp.zeros((_C, 1), jnp.float32)
    s1 = jnp.zeros((_C, 1), jnp.float32)
    for o, w in _CHUNKS:
        xc = x_ref[:, pl.ds(o, w)]
        s0 = s0 + jnp.sum(xc, axis=1, keepdims=True)
        s1 = s1 + jnp.sum(xc * xc, axis=1, keepdims=True)

    mx = jnp.full((_CL, 1), _NEG, jnp.float32)
    se = jnp.zeros((_CL, 1), jnp.float32)
    u = jnp.zeros((_CL, _C), jnp.float32)
    for o, w in _CHUNKS:
        xc = x_ref[:, pl.ds(o, w)]
        h = _affine(xc, s0, s1, gd_ref, bd_ref)
        e = jnp.dot(wd_ref[...], h, preferred_element_type=jnp.float32)
        m_new = jnp.maximum(mx, jnp.max(e, axis=1, keepdims=True))
        sc = jnp.exp(mx - m_new)
        p = jnp.exp(e - m_new)
        se = se * sc + jnp.sum(p, axis=1, keepdims=True)
        u = u * sc + jax.lax.dot_general(
            p, xc, (((1,), (1,)), ((), ())),
            preferred_element_type=jnp.float32)
        mx = m_new
    xv = u / se

    g_inner = jax.lax.dot_general(
        xv, xv, (((0,), (0,)), ((), ())), preferred_element_type=jnp.float32)
    rows = jax.lax.broadcasted_iota(jnp.int32, (_C, _C), 0)
    cols = jax.lax.broadcasted_iota(jnp.int32, (_C, _C), 1)
    eye = (rows == cols).astype(jnp.float32)
    d_col = jnp.sum(g_inner * eye, axis=1, keepdims=True)
    d_row = jnp.sum(xv * xv, axis=0, keepdims=True)
    pd = 2.0 * g_inner - d_col - d_row
    p_mat = jax.lax.dot_general(
        xv, m1_ref[...], (((0,), (0,)), ((), ())),
        preferred_element_type=jnp.float32)
    q_mat = jax.lax.dot_general(
        xv, a2_ref[...], (((0,), (0,)), ((), ())),
        preferred_element_type=jnp.float32)
    b1 = b1_ref[...]
    b2 = b2_ref[...]
    work = pd
    gmax = jnp.full((_C, _CL), _NEG, jnp.float32)
    for _ in range(_K):
        m = jnp.max(work, axis=1, keepdims=True)
        cand = jnp.where(work == m, cols, jnp.int32(1 << 30))
        sel = jnp.min(cand, axis=1, keepdims=True)
        hit = cols == sel
        onehot = hit.astype(jnp.float32)
        f_q = jnp.dot(onehot, q_mat, preferred_element_type=jnp.float32)
        g1 = jnp.maximum(p_mat - f_q + b1, 0.0)
        g2 = jnp.maximum(
            jnp.dot(g1, wg2t_ref[...], preferred_element_type=jnp.float32)
            + b2, 0.0)
        gmax = jnp.maximum(gmax, g2)
        work = jnp.where(hit, _NEG, work)
    a_mat = jnp.dot(ws2_ref[...], gmax,
                    preferred_element_type=jnp.float32)

    for o, w in _CHUNKS:
        xc = x_ref[:, pl.ds(o, w)]
        h2 = _affine(xc, s0, s1, gu_ref, beu_ref)
        e2 = jnp.dot(wu_ref[...], h2, preferred_element_type=jnp.float32)
        e2 = e2 + bu_ref[...]
        p2 = jnp.exp(e2 - jnp.max(e2, axis=0, keepdims=True))
        s2n = p2 / jnp.sum(p2, axis=0, keepdims=True)
        out_ref[:, pl.ds(o, w)] = (
            jnp.dot(ws1_ref[...], xc, preferred_element_type=jnp.float32)
            + jnp.dot(a_mat, s2n, preferred_element_type=jnp.float32)
            + bs_ref[...])


def kernel(data, bn_d_gamma, bn_d_beta, W_d, b_d, bn_u_gamma, bn_u_beta, W_u,
           b_u, W_g1, b_g1, bn_g1_gamma, bn_g1_beta, W_g2, b_g2, bn_g2_gamma,
           bn_g2_beta, W_s, b_s):
    f32 = jnp.float32
    x3 = jnp.reshape(data, (_B * _C, _N))

    s1 = bn_g1_gamma * _BN_S
    s2 = bn_g2_gamma * _BN_S
    wg1t = (W_g1 * s1[:, None]).T
    b1row = (b_g1 * s1 + bn_g1_beta)[None, :]
    wg2t = (W_g2 * s2[:, None]).T
    b2row = (b_g2 * s2 + bn_g2_beta)[None, :]
    m1 = wg1t[:_CL] + wg1t[_CL:]
    a2 = wg1t[_CL:]

    def full(shape):
        nd = len(shape)
        return pl.BlockSpec(shape, lambda b, _nd=nd: (0,) * _nd)

    tile_spec = pl.BlockSpec((_C, _N), lambda b: (b, 0))

    outp = pl.pallas_call(
        _body,
        grid=(_B,),
        in_specs=[
            tile_spec,
            full((_C, 1)), full((_C, 1)), full((_C, 1)), full((_C, 1)),
            full((_CL, _C)), full((_CL, _C)), full((_CL, 1)),
            full((_CL, _CL)), full((_CL, _CL)), full((_CL, _CL)),
            full((1, _CL)), full((1, _CL)),
            full((_C, _C)), full((_C, _C)), full((_C, 1)),
        ],
        out_specs=tile_spec,
        out_shape=jax.ShapeDtypeStruct((_B * _C, _N), f32),
        compiler_params=pltpu.CompilerParams(
            dimension_semantics=("arbitrary",)),
    )(x3,
      bn_d_gamma[:, None], bn_d_beta[:, None],
      bn_u_gamma[:, None], bn_u_beta[:, None],
      W_d, W_u, b_u[:, None],
      m1, a2, wg2t, b1row, b2row,
      W_s[:, :_C], W_s[:, _C:], b_s[:, None])

    return jnp.reshape(outp, (_B, _C, _N, 1))


# speedup vs baseline: 1.3586x; 1.3586x over previous
import jax
import jax.numpy as jnp
from jax.experimental import pallas as pl
from jax.experimental.pallas import tpu as pltpu

_B, _C, _N, _CL, _K = 4, 128, 10000, 256, 6
_NEG = float("-inf")
_BN_S = 0.9999950000374997

_TC = 1024
_CHUNKS = [(o, min(_TC, _N - o)) for o in range(0, _N, _TC)]


def _affine(x, s0, s1, g_ref, be_ref):
    mean = s0 * (1.0 / _N)
    var = s1 * (1.0 / _N) - mean * mean
    a = g_ref[...] * _BN_S * jax.lax.rsqrt(var + 1e-3)
    d = be_ref[...] - a * mean
    return jnp.maximum(a * x + d, 0.0)


def _body(x_ref, gd_ref, bd_ref, gu_ref, beu_ref, wdt_ref, wut_ref, bu_ref,
          m1_ref, a2_ref, wg2t_ref, b1_ref, b2_ref, ws1t_ref, ws2_ref,
          bs_ref, out_ref):
    s0 = jnp.zeros((1, _C), jnp.float32)
    s1 = jnp.zeros((1, _C), jnp.float32)
    for o, w in _CHUNKS:
        xc = x_ref[0, pl.ds(o, w), :]
        s0 = s0 + jnp.sum(xc, axis=0, keepdims=True)
        s1 = s1 + jnp.sum(xc * xc, axis=0, keepdims=True)

    mx = jnp.full((1, _CL), _NEG, jnp.float32)
    se = jnp.zeros((1, _CL), jnp.float32)
    u = jnp.zeros((_C, _CL), jnp.float32)
    for o, w in _CHUNKS:
        xc = x_ref[0, pl.ds(o, w), :]
        h = _affine(xc, s0, s1, gd_ref, bd_ref)
        e = jnp.dot(h, wdt_ref[...], preferred_element_type=jnp.float32)
        m_new = jnp.maximum(mx, jnp.max(e, axis=0, keepdims=True))
        sc = jnp.exp(mx - m_new)
        p = jnp.exp(e - m_new)
        se = se * sc + jnp.sum(p, axis=0, keepdims=True)
        u = u * sc + jax.lax.dot_general(
            xc, p, (((0,), (0,)), ((), ())),
            preferred_element_type=jnp.float32)
        mx = m_new
    xt = u / se

    g_inner = jax.lax.dot_general(
        xt, xt, (((1,), (1,)), ((), ())), preferred_element_type=jnp.float32)
    rows = jax.lax.broadcasted_iota(jnp.int32, (_C, _C), 0)
    cols = jax.lax.broadcasted_iota(jnp.int32, (_C, _C), 1)
    eye = (rows == cols).astype(jnp.float32)
    d_col = jnp.sum(g_inner * eye, axis=1, keepdims=True)
    d_row = jnp.sum(g_inner * eye, axis=0, keepdims=True)
    pd = 2.0 * g_inner - d_col - d_row
    p_mat = jnp.dot(xt, m1_ref[...], preferred_element_type=jnp.float32)
    q_mat = jnp.dot(xt, a2_ref[...], preferred_element_type=jnp.float32)
    b1 = b1_ref[...]
    b2 = b2_ref[...]
    work = pd
    gmax = jnp.full((_C, _CL), _NEG, jnp.float32)
    for _ in range(_K):
        m = jnp.max(work, axis=1, keepdims=True)
        cand = jnp.where(work == m, cols, jnp.int32(1 << 30))
        sel = jnp.min(cand, axis=1, keepdims=True)
        hit = cols == sel
        onehot = hit.astype(jnp.float32)
        f_q = jnp.dot(onehot, q_mat, preferred_element_type=jnp.float32)
        g1 = jnp.maximum(p_mat - f_q + b1, 0.0)
        g2 = jnp.maximum(
            jnp.dot(g1, wg2t_ref[...], preferred_element_type=jnp.float32)
            + b2, 0.0)
        gmax = jnp.maximum(gmax, g2)
        work = jnp.where(hit, _NEG, work)
    a_mat = jax.lax.dot_general(
        gmax, ws2_ref[...], (((0,), (1,)), ((), ())),
        preferred_element_type=jnp.float32)

    for o, w in _CHUNKS:
        xc = x_ref[0, pl.ds(o, w), :]
        h2 = _affine(xc, s0, s1, gu_ref, beu_ref)
        e2 = jnp.dot(h2, wut_ref[...], preferred_element_type=jnp.float32)
        e2 = e2 + bu_ref[...]
        p2 = jnp.exp(e2 - jnp.max(e2, axis=1, keepdims=True))
        s2n = p2 / jnp.sum(p2, axis=1, keepdims=True)
        out_ref[0, pl.ds(o, w), :] = (
            jnp.dot(xc, ws1t_ref[...], preferred_element_type=jnp.float32)
            + jnp.dot(s2n, a_mat, preferred_element_type=jnp.float32)
            + bs_ref[...])


def kernel(data, bn_d_gamma, bn_d_beta, W_d, b_d, bn_u_gamma, bn_u_beta, W_u,
           b_u, W_g1, b_g1, bn_g1_gamma, bn_g1_beta, W_g2, b_g2, bn_g2_gamma,
           bn_g2_beta, W_s, b_s):
    f32 = jnp.float32
    x3 = jnp.transpose(data[..., 0], (0, 2, 1))

    s1 = bn_g1_gamma * _BN_S
    s2 = bn_g2_gamma * _BN_S
    wg1t = (W_g1 * s1[:, None]).T
    b1row = (b_g1 * s1 + bn_g1_beta)[None, :]
    wg2t = (W_g2 * s2[:, None]).T
    b2row = (b_g2 * s2 + bn_g2_beta)[None, :]
    m1 = wg1t[:_CL] + wg1t[_CL:]
    a2 = wg1t[_CL:]

    def full(shape):
        nd = len(shape)
        return pl.BlockSpec(shape, lambda b, _nd=nd: (0,) * _nd)

    tile_spec = pl.BlockSpec((1, _N, _C), lambda b: (b, 0, 0))

    outp = pl.pallas_call(
        _body,
        grid=(_B,),
        in_specs=[
            tile_spec,
            full((1, _C)), full((1, _C)), full((1, _C)), full((1, _C)),
            full((_C, _CL)), full((_C, _CL)), full((1, _CL)),
            full((_CL, _CL)), full((_CL, _CL)), full((_CL, _CL)),
            full((1, _CL)), full((1, _CL)),
            full((_C, _C)), full((_C, _C)), full((1, _C)),
        ],
        out_specs=tile_spec,
        out_shape=jax.ShapeDtypeStruct((_B, _N, _C), f32),
        compiler_params=pltpu.CompilerParams(
            dimension_semantics=("arbitrary",)),
    )(x3,
      bn_d_gamma[None, :], bn_d_beta[None, :],
      bn_u_gamma[None, :], bn_u_beta[None, :],
      W_d.T, W_u.T, b_u[None, :],
      m1, a2, wg2t, b1row, b2row,
      W_s[:, :_C].T, W_s[:, _C:], b_s[None, :])

    return jnp.transpose(outp, (0, 2, 1))[..., None]


# speedup vs baseline: 1.5925x; 1.1722x over previous
import jax
import jax.numpy as jnp
from jax.experimental import pallas as pl
from jax.experimental.pallas import tpu as pltpu

_B, _C, _N, _CL, _K = 4, 128, 10000, 256, 6
_NEG = float("-inf")
_BN_S = 0.9999950000374997

_TC = 4096
_CHUNKS = [(o, min(_TC, _N - o)) for o in range(0, _N, _TC)]


def _affine(x, s0, s1, g_ref, be_ref):
    mean = s0 * (1.0 / _N)
    var = s1 * (1.0 / _N) - mean * mean
    a = g_ref[...] * _BN_S * jax.lax.rsqrt(var + 1e-3)
    d = be_ref[...] - a * mean
    return jnp.maximum(a * x + d, 0.0)


def _body(x_ref, gd_ref, bd_ref, gu_ref, beu_ref, wdt_ref, wut_ref, bu_ref,
          m1_ref, a2_ref, wg2t_ref, b1_ref, b2_ref, ws1t_ref, ws2_ref,
          bs_ref, out_ref):
    s0 = jnp.zeros((1, _C), jnp.float32)
    s1 = jnp.zeros((1, _C), jnp.float32)
    for o, w in _CHUNKS:
        xc = x_ref[0, pl.ds(o, w), :]
        s0 = s0 + jnp.sum(xc, axis=0, keepdims=True)
        s1 = s1 + jnp.sum(xc * xc, axis=0, keepdims=True)

    mx = jnp.full((1, _CL), _NEG, jnp.float32)
    se = jnp.zeros((1, _CL), jnp.float32)
    u = jnp.zeros((_C, _CL), jnp.float32)
    for o, w in _CHUNKS:
        xc = x_ref[0, pl.ds(o, w), :]
        h = _affine(xc, s0, s1, gd_ref, bd_ref)
        e = jnp.dot(h, wdt_ref[...], preferred_element_type=jnp.float32)
        m_new = jnp.maximum(mx, jnp.max(e, axis=0, keepdims=True))
        sc = jnp.exp(mx - m_new)
        p = jnp.exp(e - m_new)
        se = se * sc + jnp.sum(p, axis=0, keepdims=True)
        u = u * sc + jax.lax.dot_general(
            xc, p, (((0,), (0,)), ((), ())),
            preferred_element_type=jnp.float32)
        mx = m_new
    xt = u / se

    g_inner = jax.lax.dot_general(
        xt, xt, (((1,), (1,)), ((), ())), preferred_element_type=jnp.float32)
    rows = jax.lax.broadcasted_iota(jnp.int32, (_C, _C), 0)
    cols = jax.lax.broadcasted_iota(jnp.int32, (_C, _C), 1)
    eye = (rows == cols).astype(jnp.float32)
    d_col = jnp.sum(g_inner * eye, axis=1, keepdims=True)
    d_row = jnp.sum(g_inner * eye, axis=0, keepdims=True)
    pd = 2.0 * g_inner - d_col - d_row
    p_mat = jnp.dot(xt, m1_ref[...], preferred_element_type=jnp.float32)
    q_mat = jnp.dot(xt, a2_ref[...], preferred_element_type=jnp.float32)
    b1 = b1_ref[...]
    b2 = b2_ref[...]
    work = pd
    gmax = jnp.full((_C, _CL), _NEG, jnp.float32)
    for _ in range(_K):
        m = jnp.max(work, axis=1, keepdims=True)
        cand = jnp.where(work == m, cols, jnp.int32(1 << 30))
        sel = jnp.min(cand, axis=1, keepdims=True)
        hit = cols == sel
        onehot = hit.astype(jnp.float32)
        f_q = jnp.dot(onehot, q_mat, preferred_element_type=jnp.float32)
        g1 = jnp.maximum(p_mat - f_q + b1, 0.0)
        g2 = jnp.maximum(
            jnp.dot(g1, wg2t_ref[...], preferred_element_type=jnp.float32)
            + b2, 0.0)
        gmax = jnp.maximum(gmax, g2)
        work = jnp.where(hit, _NEG, work)
    a_mat = jax.lax.dot_general(
        gmax, ws2_ref[...], (((0,), (1,)), ((), ())),
        preferred_element_type=jnp.float32)

    for o, w in _CHUNKS:
        xc = x_ref[0, pl.ds(o, w), :]
        h2 = _affine(xc, s0, s1, gu_ref, beu_ref)
        e2 = jnp.dot(h2, wut_ref[...], preferred_element_type=jnp.float32)
        e2 = e2 + bu_ref[...]
        p2 = jnp.exp(e2 - jnp.max(e2, axis=1, keepdims=True))
        s2n = p2 / jnp.sum(p2, axis=1, keepdims=True)
        out_ref[0, pl.ds(o, w), :] = (
            jnp.dot(xc, ws1t_ref[...], preferred_element_type=jnp.float32)
            + jnp.dot(s2n, a_mat, preferred_element_type=jnp.float32)
            + bs_ref[...])


def kernel(data, bn_d_gamma, bn_d_beta, W_d, b_d, bn_u_gamma, bn_u_beta, W_u,
           b_u, W_g1, b_g1, bn_g1_gamma, bn_g1_beta, W_g2, b_g2, bn_g2_gamma,
           bn_g2_beta, W_s, b_s):
    f32 = jnp.float32
    x3 = jnp.transpose(data[..., 0], (0, 2, 1))

    s1 = bn_g1_gamma * _BN_S
    s2 = bn_g2_gamma * _BN_S
    wg1t = (W_g1 * s1[:, None]).T
    b1row = (b_g1 * s1 + bn_g1_beta)[None, :]
    wg2t = (W_g2 * s2[:, None]).T
    b2row = (b_g2 * s2 + bn_g2_beta)[None, :]
    m1 = wg1t[:_CL] + wg1t[_CL:]
    a2 = wg1t[_CL:]

    def full(shape):
        nd = len(shape)
        return pl.BlockSpec(shape, lambda b, _nd=nd: (0,) * _nd)

    tile_spec = pl.BlockSpec((1, _N, _C), lambda b: (b, 0, 0))

    outp = pl.pallas_call(
        _body,
        grid=(_B,),
        in_specs=[
            tile_spec,
            full((1, _C)), full((1, _C)), full((1, _C)), full((1, _C)),
            full((_C, _CL)), full((_C, _CL)), full((1, _CL)),
            full((_CL, _CL)), full((_CL, _CL)), full((_CL, _CL)),
            full((1, _CL)), full((1, _CL)),
            full((_C, _C)), full((_C, _C)), full((1, _C)),
        ],
        out_specs=tile_spec,
        out_shape=jax.ShapeDtypeStruct((_B, _N, _C), f32),
        compiler_params=pltpu.CompilerParams(
            dimension_semantics=("arbitrary",)),
    )(x3,
      bn_d_gamma[None, :], bn_d_beta[None, :],
      bn_u_gamma[None, :], bn_u_beta[None, :],
      W_d.T, W_u.T, b_u[None, :],
      m1, a2, wg2t, b1row, b2row,
      W_s[:, :_C].T, W_s[:, _C:], b_s[None, :])

    return jnp.transpose(outp, (0, 2, 1))[..., None]
